# Initial kernel scaffold; baseline (speedup 1.0000x reference)
import jax
import jax.numpy as jnp
from jax.experimental import pallas as pl


def kernel(h, edge_index, idx, W, b):
    raise NotImplementedError("write your pallas kernel here")


# speedup vs baseline: 8.8060x; 8.8060x over previous
import functools

import jax
import jax.numpy as jnp
from jax import lax
from jax.experimental import pallas as pl
from jax.experimental.pallas import tpu as pltpu
from jax.experimental.pallas import tpu_sc as plsc

NC = 2
NS = 16
NW = NC * NS
CHUNK = 128
CNT_W = 16
TRASH = 240

_SC_PARAMS = pltpu.CompilerParams(needs_layout_passes=False)


@functools.cache
def _sc_mesh():
    return plsc.VectorSubcoreMesh(
        core_axis_name="c", subcore_axis_name="s",
        num_cores=NC, num_subcores=NS)


def _sc_sums(h_pad, srcm, dstm, zeros_hbm):
    npad, d = h_pad.shape
    rpw = srcm.shape[0] // NW
    rpt = npad // NS

    @functools.partial(
        pl.kernel,
        out_type=jax.ShapeDtypeStruct((NC, npad, d), jnp.float32),
        mesh=_sc_mesh(),
        scratch_types=[
            pltpu.VMEM((CHUNK,), jnp.int32),
            pltpu.VMEM((CHUNK,), jnp.int32),
            pltpu.VMEM((CHUNK,), jnp.int32),
            pltpu.VMEM((CHUNK,), jnp.int32),
            pltpu.VMEM((CHUNK, d), jnp.float32),
            pltpu.VMEM((CHUNK, d), jnp.float32),
            pltpu.VMEM_SHARED((npad, d), jnp.float32),
            pltpu.SemaphoreType.DMA,
            pltpu.SemaphoreType.DMA,
            pltpu.SemaphoreType.DMA,
            pltpu.SemaphoreType.DMA,
            pltpu.SemaphoreType.DMA,
            pltpu.SemaphoreType.DMA,
        ],
        compiler_params=_SC_PARAMS,
    )
    def body(h_hbm, src_hbm, dst_hbm, z_hbm, sum_out,
             sidx0, sidx1, didx0, didx1, rows0, rows1, ssum,
             sem_g0, sem_g1, sem_d0, sem_d1, sem_s0, sem_s1):
        cid = lax.axis_index("c")
        sid = lax.axis_index("s")
        wid = sid * NC + cid
        r0 = sid * rpt
        base = wid * rpw

        pltpu.sync_copy(z_hbm, rows0)
        for k in range(rpt // CHUNK):
            pltpu.sync_copy(rows0, ssum.at[pl.ds(r0 + k * CHUNK, CHUNK)])
        plsc.subcore_barrier()

        def scopy(j, sbuf, sem):
            return pltpu.make_async_copy(src_hbm.at[base + j], sbuf, sem)

        def dcopy(j, dbuf, sem):
            return pltpu.make_async_copy(dst_hbm.at[base + j], dbuf, sem)

        def gcopy(sbuf, rows, sem):
            return pltpu.make_async_copy(h_hbm.at[sbuf], rows, sem)

        gfull = rpw // 2
        scopy(0, sidx0, sem_s0).start()
        scopy(1, sidx1, sem_s1).start()
        dcopy(0, didx0, sem_d0).start()
        scopy(0, sidx0, sem_s0).wait()
        gcopy(sidx0, rows0, sem_g0).start()

        @pl.loop(0, gfull)
        def _(g):
            j = 2 * g
            scopy(j + 1, sidx1, sem_s1).wait()
            gcopy(sidx1, rows1, sem_g1).start()
            dcopy(j + 1, didx1, sem_d1).start()
            gcopy(sidx0, rows0, sem_g0).wait()

            @pl.when(g + 1 < gfull)
            def _():
                scopy(j + 2, sidx0, sem_s0).start()

            dcopy(j, didx0, sem_d0).wait()
            pltpu.sync_copy(rows0, ssum.at[didx0], add=True)

            @pl.when(g + 1 < gfull)
            def _():
                dcopy(j + 2, didx0, sem_d0).start()

            gcopy(sidx1, rows1, sem_g1).wait()

            @pl.when(g + 1 < gfull)
            def _():
                scopy(j + 2, sidx0, sem_s0).wait()
                gcopy(sidx0, rows0, sem_g0).start()
                scopy(j + 3, sidx1, sem_s1).start()

            dcopy(j + 1, didx1, sem_d1).wait()
            pltpu.sync_copy(rows1, ssum.at[didx1], add=True)

        plsc.subcore_barrier()

        for k in range(rpt // CHUNK):
            o = r0 + k * CHUNK
            pltpu.sync_copy(ssum.at[pl.ds(o, CHUNK)], rows0)
            pltpu.sync_copy(rows0, sum_out.at[cid, pl.ds(o, CHUNK)])

    return body(h_pad, srcm, dstm, zeros_hbm)


def _sc_counts(dstm, npad, d, ones_hbm_in, zeros_hbm_in):
    rpw = dstm.shape[0] // NW
    rpt = npad // NS

    @functools.partial(
        pl.kernel,
        out_type=jax.ShapeDtypeStruct((NC, npad, d), jnp.float32),
        mesh=_sc_mesh(),
        scratch_types=[
            pltpu.VMEM((CHUNK,), jnp.int32),
            pltpu.VMEM((CHUNK, d), jnp.float32),
            pltpu.VMEM((CHUNK, d), jnp.float32),
            pltpu.VMEM_SHARED((npad, d), jnp.float32),
        ],
    )
    def body(dst_hbm, ones_hbm, zeros_hbm, cnt_out, didx, ones_v, cstage,
             scnt):
        cid = lax.axis_index("c")
        sid = lax.axis_index("s")
        wid = sid * NC + cid
        r0 = sid * rpt

        pltpu.sync_copy(ones_hbm, ones_v)
        pltpu.sync_copy(zeros_hbm, cstage)
        for k in range(rpt // CHUNK):
            pltpu.sync_copy(cstage, scnt.at[pl.ds(r0 + k * CHUNK, CHUNK)])
        plsc.subcore_barrier()

        base = wid * rpw

        @pl.loop(0, rpw)
        def _(j):
            pltpu.sync_copy(dst_hbm.at[base + j], didx)
            pltpu.sync_copy(ones_v, scnt.at[didx], add=True)

        plsc.subcore_barrier()

        for k in range(rpt // CHUNK):
            o = r0 + k * CHUNK
            pltpu.sync_copy(scnt.at[pl.ds(o, CHUNK)], cstage)
            pltpu.sync_copy(cstage, cnt_out.at[cid, pl.ds(o, CHUNK)])

    return body(dstm, ones_hbm_in, zeros_hbm_in)


def _combine_body(h_ref, p_ref, c_ref, w1_ref, w2_ref, b_ref, o_ref):
    s = p_ref[0] + p_ref[1]
    cnt = c_ref[0, :, :1] + c_ref[1, :, :1]
    hn = s / jnp.maximum(cnt, 1.0)
    o_ref[...] = (
        jnp.dot(h_ref[...], w1_ref[...], preferred_element_type=jnp.float32)
        + jnp.dot(hn, w2_ref[...], preferred_element_type=jnp.float32)
        + b_ref[...])


def _tc_combine(h_pad, psum, pcnt, w1, w2, b2d):
    npad, d = h_pad.shape
    dout = w1.shape[1]
    blk = 1280
    grid = npad // blk
    return pl.pallas_call(
        _combine_body,
        grid=(grid,),
        in_specs=[
            pl.BlockSpec((blk, d), lambda i: (i, 0)),
            pl.BlockSpec((NC, blk, d), lambda i: (0, i, 0)),
            pl.BlockSpec((NC, blk, d), lambda i: (0, i, 0)),
            pl.BlockSpec((d, dout), lambda i: (0, 0)),
            pl.BlockSpec((d, dout), lambda i: (0, 0)),
            pl.BlockSpec((1, dout), lambda i: (0, 0)),
        ],
        out_specs=pl.BlockSpec((blk, dout), lambda i: (i, 0)),
        out_shape=jax.ShapeDtypeStruct((npad, dout), jnp.float32),
    )(h_pad, psum, pcnt, w1, w2, b2d)


def kernel(h, edge_index, idx, W, b):
    del idx
    n, d = h.shape
    e = edge_index.shape[1]
    src = edge_index[0].astype(jnp.int32)
    dst = edge_index[1].astype(jnp.int32)

    npad = -(-(n + TRASH) // (CHUNK * NS)) * (CHUNK * NS)
    h_pad = jnp.zeros((npad, d), jnp.float32).at[:n].set(h)

    epad = -(-e // (CHUNK * NW)) * (CHUNK * NW)
    pad = epad - e
    spread = n + (jnp.arange(pad, dtype=jnp.int32) % TRASH)
    srcm = jnp.concatenate([src, spread]).reshape(-1, CHUNK)
    dstm = jnp.concatenate([dst, spread]).reshape(-1, CHUNK)

    zeros_hbm = jnp.zeros((CHUNK, d), jnp.float32)
    ones_hbm = jnp.ones((CHUNK, d), jnp.float32)
    psum = _sc_sums(h_pad, srcm, dstm, zeros_hbm)
    pcnt = _sc_counts(dstm, npad, d, ones_hbm, zeros_hbm)
    w1 = W[:, :d].T
    w2 = W[:, d:].T
    out = _tc_combine(h_pad, psum, pcnt, w1, w2, b.reshape(1, -1))
    return out[:n]
